# baseline (speedup 1.0000x reference)
import jax
import jax.numpy as jnp
from jax.experimental import pallas as pl

NUM_CLASSES = 20
OH_ROWS = 32

BLK = 16384


NSPLIT = 2


def _partial_proto(f, lab):
    blk = f.shape[-1]
    sumsq = jnp.sum(f * f, axis=0, keepdims=True)
    invn = 1.0 / jnp.maximum(jnp.sqrt(sumsq), 1e-12)

    kiota = jax.lax.broadcasted_iota(jnp.int32, (OH_ROWS, blk), 0)
    oh = jnp.where(kiota == lab, invn, 0.0)

    return jax.lax.dot_general(
        f, oh, (((1,), (1,)), ((), ())),
        preferred_element_type=jnp.float32)


def _proto_body(nsteps):
    def body(*refs):
        o_ref = refs[-1]
        f_refs = refs[:NSPLIT]
        l_refs = refs[NSPLIT:2 * NSPLIT]

        step = pl.program_id(0) * pl.num_programs(1) + pl.program_id(1)

        partial = _partial_proto(f_refs[0][0], l_refs[0][0])
        for s in range(1, NSPLIT):
            partial += _partial_proto(f_refs[s][0], l_refs[s][0])

        @pl.when(step == 0)
        def _():
            o_ref[...] = jnp.zeros_like(o_ref)

        o_ref[...] += partial

        @pl.when(step == nsteps - 1)
        def _():
            p = o_ref[...]
            pn = jnp.sqrt(jnp.sum(p * p, axis=0, keepdims=True))
            o_ref[...] = p / jnp.maximum(pn, 1e-12)

    return body


def kernel(features, labels):
    bs, c, h, w = features.shape
    hw = h * w
    feats = features.reshape(bs, c, hw)
    lab = labels.astype(jnp.int32).reshape(bs, 1, hw)

    bsp = bs // NSPLIT
    nj = hw // BLK
    nsteps = bsp * nj

    f_parts = [feats[s * bsp:(s + 1) * bsp] for s in range(NSPLIT)]
    l_parts = [lab[s * bsp:(s + 1) * bsp] for s in range(NSPLIT)]

    out = pl.pallas_call(
        _proto_body(nsteps),
        grid=(bsp, nj),
        in_specs=(
            [pl.BlockSpec((1, c, BLK), lambda b, j: (b, 0, j))
             for _ in range(NSPLIT)]
            + [pl.BlockSpec((1, 1, BLK), lambda b, j: (b, 0, j))
               for _ in range(NSPLIT)]
        ),
        out_specs=pl.BlockSpec((c, OH_ROWS), lambda b, j: (0, 0)),
        out_shape=jax.ShapeDtypeStruct((c, OH_ROWS), jnp.float32),
    )(*f_parts, *l_parts)

    return out[:, :NUM_CLASSES - 1]


# speedup vs baseline: 1.8879x; 1.8879x over previous
import jax
import jax.numpy as jnp
from jax.experimental import pallas as pl
from jax.experimental.pallas import tpu as pltpu

NUM_CLASSES = 20
OH_ROWS = 32

CH = 4096
NBUF = 4


def _partial_proto(f, lab):
    blk = f.shape[-1]
    sumsq = jnp.sum(f * f, axis=0, keepdims=True)
    invn = 1.0 / jnp.maximum(jnp.sqrt(sumsq), 1e-12)

    kiota = jax.lax.broadcasted_iota(jnp.int32, (OH_ROWS, blk), 0)
    oh = jnp.where(kiota == lab, invn, 0.0)

    return jax.lax.dot_general(
        f, oh, (((1,), (1,)), ((), ())),
        preferred_element_type=jnp.float32)


def _make_body(nsteps, nj):
    def body(f_hbm, l_hbm, o_ref, fbuf, lbuf, fsem, lsem):
        i = pl.program_id(0)

        def start(k):
            b = k // nj
            j = k % nj
            slot = jax.lax.rem(k, NBUF)
            pltpu.make_async_copy(
                f_hbm.at[b, :, pl.ds(j * CH, CH)],
                fbuf.at[slot], fsem.at[slot]).start()
            pltpu.make_async_copy(
                l_hbm.at[b, :, pl.ds(j * CH, CH)],
                lbuf.at[slot], lsem.at[slot]).start()

        @pl.when(i == 0)
        def _():
            for k in range(NBUF - 1):
                start(k)

        nxt = i + NBUF - 1

        @pl.when(nxt < nsteps)
        def _():
            start(nxt)

        slot = jax.lax.rem(i, NBUF)
        b = i // nj
        j = i % nj
        pltpu.make_async_copy(
            f_hbm.at[b, :, pl.ds(j * CH, CH)],
            fbuf.at[slot], fsem.at[slot]).wait()
        pltpu.make_async_copy(
            l_hbm.at[b, :, pl.ds(j * CH, CH)],
            lbuf.at[slot], lsem.at[slot]).wait()

        partial = _partial_proto(fbuf[slot], lbuf[slot])

        @pl.when(i == 0)
        def _():
            o_ref[...] = jnp.zeros_like(o_ref)

        o_ref[...] += partial

        @pl.when(i == nsteps - 1)
        def _():
            p = o_ref[...]
            pn = jnp.sqrt(jnp.sum(p * p, axis=0, keepdims=True))
            o_ref[...] = p / jnp.maximum(pn, 1e-12)

    return body


def kernel(features, labels):
    bs, c, h, w = features.shape
    hw = h * w
    feats = features.reshape(bs, c, hw)
    lab = labels.astype(jnp.int32).reshape(bs, 1, hw)

    nj = hw // CH
    nsteps = bs * nj

    out = pl.pallas_call(
        _make_body(nsteps, nj),
        grid=(nsteps,),
        in_specs=[
            pl.BlockSpec(memory_space=pl.ANY),
            pl.BlockSpec(memory_space=pl.ANY),
        ],
        out_specs=pl.BlockSpec((c, OH_ROWS), lambda i: (0, 0)),
        out_shape=jax.ShapeDtypeStruct((c, OH_ROWS), jnp.float32),
        scratch_shapes=[
            pltpu.VMEM((NBUF, c, CH), jnp.float32),
            pltpu.VMEM((NBUF, 1, CH), jnp.int32),
            pltpu.SemaphoreType.DMA((NBUF,)),
            pltpu.SemaphoreType.DMA((NBUF,)),
        ],
    )(feats, lab)

    return out[:, :NUM_CLASSES - 1]
